# baseline (speedup 1.0000x reference)
import functools

import jax
import jax.numpy as jnp
from jax import lax
from jax.experimental import pallas as pl
from jax.experimental.pallas import tpu as pltpu
from jax.experimental.pallas import tpu_sc as plsc

_NC = 2
_NS = 16
_NW = _NC * _NS

_CHUNK = 128
_NBUF = 4

_PAD_BLK = 8000


_PAD_NB = 6
_PAD_GRID = 125


def _pad_body(t_hbm, o_hbm, ibufs, obufs, isems, osems):
    def in_copy(c):
        return pltpu.make_async_copy(
            t_hbm.at[pl.ds(c * _PAD_BLK, _PAD_BLK), :],
            ibufs.at[c % _PAD_NB], isems.at[c % _PAD_NB])

    def out_copy(c):
        return pltpu.make_async_copy(
            obufs.at[c % _PAD_NB],
            o_hbm.at[pl.ds(c * _PAD_BLK, _PAD_BLK), :],
            osems.at[c % _PAD_NB])

    for b in range(_PAD_NB):
        in_copy(b).start()
    for c in range(_PAD_GRID):
        b = c % _PAD_NB
        in_copy(c).wait()
        if c >= _PAD_NB:
            out_copy(c - _PAD_NB).wait()
        obufs[b, :, :96] = ibufs[b]
        out_copy(c).start()
        if c + _PAD_NB < _PAD_GRID:
            in_copy(c + _PAD_NB).start()
    for c in range(_PAD_GRID - _PAD_NB, _PAD_GRID):
        out_copy(c).wait()


def _tc_pad_table(table):
    v, d = table.shape
    return pl.pallas_call(
        _pad_body,
        in_specs=[pl.BlockSpec(memory_space=pl.ANY)],
        out_specs=pl.BlockSpec(memory_space=pl.ANY),
        out_shape=jax.ShapeDtypeStruct((v, 128), jnp.float32),
        scratch_shapes=[pltpu.VMEM((_PAD_NB, _PAD_BLK, d), jnp.float32),
                        pltpu.VMEM((_PAD_NB, _PAD_BLK, 128), jnp.float32),
                        pltpu.SemaphoreType.DMA((_PAD_NB,)),
                        pltpu.SemaphoreType.DMA((_PAD_NB,))],
    )(table)


def _sc_gather(table_p, idx_flat):
    n = idx_flat.shape[0]
    d = table_p.shape[1]
    per_w = n // _NW
    nch = per_w // _CHUNK
    assert per_w % _CHUNK == 0 and nch % _NBUF == 0

    mesh = plsc.VectorSubcoreMesh(core_axis_name="c", subcore_axis_name="s")

    @functools.partial(
        pl.kernel,
        out_type=jax.ShapeDtypeStruct((n, d), jnp.float32),
        mesh=mesh,
        scratch_types=[
            pltpu.VMEM((_NBUF, _CHUNK), jnp.int32),
            pltpu.VMEM((_NBUF, _CHUNK, d), jnp.float32),
        ] + [pltpu.SemaphoreType.DMA] * _NBUF,
    )
    def gather_kernel(table_hbm, idx_hbm, out_hbm, idx_v, rows_v, *sems):
        wid = lax.axis_index("s") * _NC + lax.axis_index("c")
        base = wid * per_w

        def load_idx(b, j):
            pltpu.sync_copy(idx_hbm.at[pl.ds(base + j * _CHUNK, _CHUNK)],
                            idx_v.at[b])

        def start_gather(b):
            pltpu.async_copy(table_hbm.at[idx_v.at[b]], rows_v.at[b], sems[b])

        def wait_gather(b):
            pltpu.make_async_copy(table_hbm.at[idx_v.at[b]], rows_v.at[b],
                                  sems[b]).wait()

        def store_rows(b, j):
            pltpu.sync_copy(rows_v.at[b],
                            out_hbm.at[pl.ds(base + j * _CHUNK, _CHUNK)])

        for b in range(_NBUF):
            load_idx(b, b)
            start_gather(b)

        @pl.loop(0, nch - _NBUF, step=_NBUF)
        def _(j0):
            for b in range(_NBUF):
                j = j0 + b
                wait_gather(b)
                store_rows(b, j)
                load_idx(b, j + _NBUF)
                start_gather(b)

        for b in range(_NBUF):
            wait_gather(b)
            store_rows(b, nch - _NBUF + b)

    return gather_kernel(table_p, idx_flat)


_BLK = 4096


def _mlp_body(g_ref, pos_ref, w1_ref, b1_ref, w2_ref, b2_ref, o_ref):
    h = (g_ref[:, :96] + pos_ref[...]).astype(jnp.bfloat16)
    h1 = jnp.dot(h, w1_ref[...], preferred_element_type=jnp.float32)
    h1 = jnp.maximum(h1 + b1_ref[...], 0.0).astype(jnp.bfloat16)
    o = jnp.dot(h1, w2_ref[...], preferred_element_type=jnp.float32)
    o_ref[...] = o + b2_ref[...]


def _tc_mlp(g, pos_rep, w1, b1, w2, b2):
    n, dp = g.shape
    d = w2.shape[1]
    inner = w1.shape[1]
    grid = (n // _BLK,)
    return pl.pallas_call(
        _mlp_body,
        grid=grid,
        in_specs=[
            pl.BlockSpec((_BLK, dp), lambda i: (i, 0)),
            pl.BlockSpec((_BLK, 96), lambda i: (0, 0)),
            pl.BlockSpec((96, inner), lambda i: (0, 0)),
            pl.BlockSpec((1, inner), lambda i: (0, 0)),
            pl.BlockSpec((inner, d), lambda i: (0, 0)),
            pl.BlockSpec((1, d), lambda i: (0, 0)),
        ],
        out_specs=pl.BlockSpec((_BLK, d), lambda i: (i, 0)),
        out_shape=jax.ShapeDtypeStruct((n, d), jnp.float32),
        compiler_params=pltpu.CompilerParams(
            dimension_semantics=("parallel",)),
    )(g, pos_rep, w1, b1, w2, b2)


def kernel(x, ks_table, pos_table, W1, b1, W2, b2):
    batch, seq = x.shape
    d = ks_table.shape[1]
    n = batch * seq
    idx_flat = x.reshape(n).astype(jnp.int32)
    table_p = _tc_pad_table(ks_table)
    g = _sc_gather(table_p, idx_flat)
    pos_rep = jnp.tile(pos_table, (_BLK // seq, 1))
    w1_p = W1.astype(jnp.bfloat16)
    y = _tc_mlp(g, pos_rep, w1_p, b1.reshape(1, -1),
                W2.astype(jnp.bfloat16), b2.reshape(1, -1))
    return y.reshape(batch, seq, d)


# speedup vs baseline: 1.0030x; 1.0030x over previous
import functools

import jax
import jax.numpy as jnp
from jax import lax
from jax.experimental import pallas as pl
from jax.experimental.pallas import tpu as pltpu
from jax.experimental.pallas import tpu_sc as plsc

_NC = 2
_NS = 16
_NW = _NC * _NS

_CHUNK = 128
_NBUF = 4

_PAD_BLK = 8000


_PAD_NB = 6


def _pad_body(t_ref, o_hbm, bufs, sems):
    i = pl.program_id(0)

    def out_copy(buf, step):
        return pltpu.make_async_copy(
            bufs.at[buf],
            o_hbm.at[pl.ds(step * _PAD_BLK, _PAD_BLK), :],
            sems.at[buf])

    @pl.when(i >= _PAD_NB)
    def _():
        out_copy(i % _PAD_NB, i - _PAD_NB).wait()

    bufs[i % _PAD_NB, :, :96] = t_ref[...]
    out_copy(i % _PAD_NB, i).start()

    @pl.when(i == _PAD_GRID - 1)
    def _():
        for c in range(_PAD_GRID - _PAD_NB, _PAD_GRID):
            out_copy(c % _PAD_NB, c).wait()


_PAD_GRID = 125


def _tc_pad_table(table):
    v, d = table.shape
    return pl.pallas_call(
        _pad_body,
        grid=(_PAD_GRID,),
        in_specs=[pl.BlockSpec((_PAD_BLK, d), lambda i: (i, 0))],
        out_specs=pl.BlockSpec(memory_space=pl.ANY),
        out_shape=jax.ShapeDtypeStruct((v, 128), jnp.float32),
        scratch_shapes=[pltpu.VMEM((_PAD_NB, _PAD_BLK, 128), jnp.float32),
                        pltpu.SemaphoreType.DMA((_PAD_NB,))],
    )(table)


def _sc_gather(table_p, idx_flat):
    n = idx_flat.shape[0]
    d = table_p.shape[1]
    per_w = n // _NW
    nch = per_w // _CHUNK
    assert per_w % _CHUNK == 0 and nch % _NBUF == 0

    mesh = plsc.VectorSubcoreMesh(core_axis_name="c", subcore_axis_name="s")

    @functools.partial(
        pl.kernel,
        out_type=jax.ShapeDtypeStruct((n, d), jnp.float32),
        mesh=mesh,
        scratch_types=[
            pltpu.VMEM((_NBUF, _CHUNK), jnp.int32),
            pltpu.VMEM((_NBUF, _CHUNK, d), jnp.float32),
        ] + [pltpu.SemaphoreType.DMA] * _NBUF,
    )
    def gather_kernel(table_hbm, idx_hbm, out_hbm, idx_v, rows_v, *sems):
        wid = lax.axis_index("s") * _NC + lax.axis_index("c")
        base = wid * per_w

        def load_idx(b, j):
            pltpu.sync_copy(idx_hbm.at[pl.ds(base + j * _CHUNK, _CHUNK)],
                            idx_v.at[b])

        def start_gather(b):
            pltpu.async_copy(table_hbm.at[idx_v.at[b]], rows_v.at[b], sems[b])

        def wait_gather(b):
            pltpu.make_async_copy(table_hbm.at[idx_v.at[b]], rows_v.at[b],
                                  sems[b]).wait()

        def store_rows(b, j):
            pltpu.sync_copy(rows_v.at[b],
                            out_hbm.at[pl.ds(base + j * _CHUNK, _CHUNK)])

        for b in range(_NBUF):
            load_idx(b, b)
            start_gather(b)

        @pl.loop(0, nch - _NBUF, step=_NBUF)
        def _(j0):
            for b in range(_NBUF):
                j = j0 + b
                wait_gather(b)
                store_rows(b, j)
                load_idx(b, j + _NBUF)
                start_gather(b)

        for b in range(_NBUF):
            wait_gather(b)
            store_rows(b, nch - _NBUF + b)

    return gather_kernel(table_p, idx_flat)


_BLK = 4096


def _mlp_body(g_ref, pos_ref, w1_ref, b1_ref, w2_ref, b2_ref, o_ref):
    h = (g_ref[:, :96] + pos_ref[...]).astype(jnp.bfloat16)
    h1 = jnp.dot(h, w1_ref[...], preferred_element_type=jnp.float32)
    h1 = jnp.maximum(h1 + b1_ref[...], 0.0).astype(jnp.bfloat16)
    o = jnp.dot(h1, w2_ref[...], preferred_element_type=jnp.float32)
    o_ref[...] = o + b2_ref[...]


def _tc_mlp(g, pos_rep, w1, b1, w2, b2):
    n, dp = g.shape
    d = w2.shape[1]
    inner = w1.shape[1]
    grid = (n // _BLK,)
    return pl.pallas_call(
        _mlp_body,
        grid=grid,
        in_specs=[
            pl.BlockSpec((_BLK, dp), lambda i: (i, 0)),
            pl.BlockSpec((_BLK, 96), lambda i: (0, 0)),
            pl.BlockSpec((96, inner), lambda i: (0, 0)),
            pl.BlockSpec((1, inner), lambda i: (0, 0)),
            pl.BlockSpec((inner, d), lambda i: (0, 0)),
            pl.BlockSpec((1, d), lambda i: (0, 0)),
        ],
        out_specs=pl.BlockSpec((_BLK, d), lambda i: (i, 0)),
        out_shape=jax.ShapeDtypeStruct((n, d), jnp.float32),
        compiler_params=pltpu.CompilerParams(
            dimension_semantics=("parallel",)),
    )(g, pos_rep, w1, b1, w2, b2)


def kernel(x, ks_table, pos_table, W1, b1, W2, b2):
    batch, seq = x.shape
    d = ks_table.shape[1]
    n = batch * seq
    idx_flat = x.reshape(n).astype(jnp.int32)
    table_p = _tc_pad_table(ks_table)
    g = _sc_gather(table_p, idx_flat)
    pos_rep = jnp.tile(pos_table, (_BLK // seq, 1))
    w1_p = W1.astype(jnp.bfloat16)
    y = _tc_mlp(g, pos_rep, w1_p, b1.reshape(1, -1),
                W2.astype(jnp.bfloat16), b2.reshape(1, -1))
    return y.reshape(batch, seq, d)
